# Initial kernel scaffold; baseline (speedup 1.0000x reference)
import jax
import jax.numpy as jnp
from jax.experimental import pallas as pl


def kernel(x, codebook, W_in, b_in, W_cat, b_cat, W1, b1, W2, b2, W_out, b_out):
    raise NotImplementedError("write your pallas kernel here")


# speedup vs baseline: 2.8692x; 2.8692x over previous
import functools

import jax
import jax.numpy as jnp
from jax import lax
from jax.experimental import pallas as pl
from jax.experimental.pallas import tpu as pltpu

_BF = jnp.bfloat16
_F32 = jnp.float32


def _bdot(a, b):
    dn = (((1,), (0,)), ((), ()))
    return lax.dot_general(a.astype(_BF), b.astype(_BF), dn,
                           preferred_element_type=_F32)


def _argmin_update(d2, cand, blk, kb, codes_ref, xh_out_ref, best_ref):
    loc_min = jnp.min(d2, axis=1, keepdims=True)
    iota = lax.broadcasted_iota(jnp.int32, d2.shape, 1)
    loc_idx = jnp.min(jnp.where(d2 == loc_min, iota, kb),
                      axis=1, keepdims=True)
    oh = (iota == loc_idx).astype(_F32)
    loc_cw = jnp.sum(cand * oh[:, :, None], axis=1)
    gidx = blk * kb + loc_idx

    @pl.when(blk == 0)
    def _init():
        best_ref[...] = loc_min
        codes_ref[...] = gidx
        xh_out_ref[...] = loc_cw

    @pl.when(blk > 0)
    def _update():
        upd = loc_min < best_ref[...]
        best_ref[...] = jnp.where(upd, loc_min, best_ref[...])
        codes_ref[...] = jnp.where(upd, gidx, codes_ref[...])
        xh_out_ref[...] = jnp.where(upd, loc_cw, xh_out_ref[...])


def _step0_body(x_ref, cb_ref, codes_ref, xhat_ref, best_ref):
    blk = pl.program_id(0)
    x = x_ref[...]
    cb = cb_ref[...]
    kb = cb.shape[0]
    bsz = x.shape[0]
    norm = jnp.sum(cb[None, :, :] * cb[None, :, :], axis=-1)
    cbr = cb.astype(_BF).astype(_F32)
    cross = jnp.sum(x[:, None, :] * cbr[None, :, :], axis=-1)
    d2 = norm - 2.0 * cross
    cand = jnp.broadcast_to(cb[None, :, :], (bsz, kb, cb.shape[1]))
    _argmin_update(d2, cand, blk, kb, codes_ref, xhat_ref, best_ref)


def _step_body(x_ref, xhat_ref, cb_ref, w_in_ref, b_in_ref, wct_ref, wcb_ref,
               b_cat_ref, w1_ref, b1_ref, w2_ref, b2_ref, w_out_ref, b_out_ref,
               codes_ref, xh_out_ref, best_ref, *, n_res):
    kb = cb_ref.shape[0]
    blk = pl.program_id(0)
    x = x_ref[...]
    xhat = xhat_ref[...]
    cb = cb_ref[...]
    bsz = x.shape[0]

    a = _bdot(_bdot(cb, w_in_ref[...]) + b_in_ref[...], wct_ref[...]) \
        + b_cat_ref[...]
    c = _bdot(xhat, wcb_ref[...])

    h = (a[None, :, :] + c[:, None, :]).reshape(bsz * kb, -1)
    for l in range(n_res):
        t = jnp.maximum(_bdot(h, w1_ref[l]) + b1_ref[l], 0.0)
        h = h + _bdot(t, w2_ref[l]) + b2_ref[l]
    out = _bdot(h, w_out_ref[...]) + b_out_ref[...]
    out3 = out.reshape(bsz, kb, -1) + cb[None, :, :] + xhat[:, None, :]

    norm = jnp.sum(out3 * out3, axis=-1)
    xr = x.astype(_BF).astype(_F32)
    our = out3.astype(_BF).astype(_F32)
    cross = jnp.sum(our * xr[:, None, :], axis=-1)
    d2 = norm - 2.0 * cross
    _argmin_update(d2, out3, blk, kb, codes_ref, xh_out_ref, best_ref)


def kernel(x, codebook, W_in, b_in, W_cat, b_cat, W1, b1, W2, b2, W_out, b_out):
    nL = W_in.shape[0]
    Mm = nL + 1
    Kk = codebook.shape[0] // Mm
    Bb, Dd = x.shape
    DH = W_in.shape[2]
    Ll = W1.shape[1]
    KB = 64
    f32 = jnp.float32

    rep2 = lambda k: (0, 0)
    rep3 = lambda k: (0, 0, 0)
    out_specs = (pl.BlockSpec((Bb, 1), rep2), pl.BlockSpec((Bb, Dd), rep2))
    out_shape = (jax.ShapeDtypeStruct((Bb, 1), jnp.int32),
                 jax.ShapeDtypeStruct((Bb, Dd), f32))
    seq = pltpu.CompilerParams(dimension_semantics=("arbitrary",))

    codes0, xhat = pl.pallas_call(
        _step0_body,
        grid=(Kk // KB,),
        in_specs=[pl.BlockSpec((Bb, Dd), rep2),
                  pl.BlockSpec((KB, Dd), lambda k: (k, 0))],
        out_specs=out_specs,
        out_shape=out_shape,
        scratch_shapes=[pltpu.VMEM((Bb, 1), f32)],
        compiler_params=seq,
    )(x, codebook[:Kk])

    step_call = pl.pallas_call(
        functools.partial(_step_body, n_res=Ll),
        grid=(Kk // KB,),
        in_specs=[
            pl.BlockSpec((Bb, Dd), rep2),
            pl.BlockSpec((Bb, Dd), rep2),
            pl.BlockSpec((KB, Dd), lambda k: (k, 0)),
            pl.BlockSpec((Dd, DH), rep2),
            pl.BlockSpec((1, DH), rep2),
            pl.BlockSpec((DH, DH), rep2),
            pl.BlockSpec((Dd, DH), rep2),
            pl.BlockSpec((1, DH), rep2),
            pl.BlockSpec((Ll, DH, DH), rep3),
            pl.BlockSpec((Ll, 1, DH), rep3),
            pl.BlockSpec((Ll, DH, DH), rep3),
            pl.BlockSpec((Ll, 1, DH), rep3),
            pl.BlockSpec((DH, Dd), rep2),
            pl.BlockSpec((1, Dd), rep2),
        ],
        out_specs=out_specs,
        out_shape=out_shape,
        scratch_shapes=[pltpu.VMEM((Bb, 1), f32)],
        compiler_params=seq,
    )

    codes_list = [codes0]
    for i in range(nL):
        cb_i = lax.dynamic_slice_in_dim(codebook, (i + 1) * Kk, Kk, 0)
        ci, xhat = step_call(
            x, xhat, cb_i,
            W_in[i], b_in[i][None, :],
            W_cat[i][:DH], W_cat[i][DH:], b_cat[i][None, :],
            W1[i], b1[i][:, None, :], W2[i], b2[i][:, None, :],
            W_out[i], b_out[i][None, :])
        codes_list.append(ci)

    codes_MB = jnp.concatenate([c.T for c in codes_list], axis=0)
    return codes_MB, xhat


# speedup vs baseline: 3.0668x; 1.0689x over previous
import functools

import jax
import jax.numpy as jnp
from jax import lax
from jax.experimental import pallas as pl
from jax.experimental.pallas import tpu as pltpu

_BF = jnp.bfloat16
_F32 = jnp.float32


def _bdot(a, b):
    dn = (((1,), (0,)), ((), ()))
    return lax.dot_general(a.astype(_BF), b.astype(_BF), dn,
                           preferred_element_type=_F32)


def _argmin_update(d2, cand, blk, kb, codes_ref, xh_out_ref, best_ref):
    loc_min = jnp.min(d2, axis=1, keepdims=True)
    iota = lax.broadcasted_iota(jnp.int32, d2.shape, 1)
    loc_idx = jnp.min(jnp.where(d2 == loc_min, iota, kb),
                      axis=1, keepdims=True)
    oh = (iota == loc_idx).astype(_F32)
    loc_cw = jnp.sum(cand * oh[:, :, None], axis=1)
    gidx = blk * kb + loc_idx

    c3 = len(codes_ref.shape) == 3

    @pl.when(blk == 0)
    def _init():
        best_ref[...] = loc_min
        if c3:
            codes_ref[0] = gidx
        else:
            codes_ref[...] = gidx
        xh_out_ref[...] = loc_cw

    @pl.when(blk > 0)
    def _update():
        upd = loc_min < best_ref[...]
        best_ref[...] = jnp.where(upd, loc_min, best_ref[...])
        if c3:
            codes_ref[0] = jnp.where(upd, gidx, codes_ref[0])
        else:
            codes_ref[...] = jnp.where(upd, gidx, codes_ref[...])
        xh_out_ref[...] = jnp.where(upd, loc_cw, xh_out_ref[...])


def _step0_body(x_ref, cb_ref, codes_ref, xhat_ref, best_ref):
    blk = pl.program_id(0)
    x = x_ref[...]
    cb = cb_ref[...]
    kb = cb.shape[0]
    bsz = x.shape[0]
    norm = jnp.sum(cb[None, :, :] * cb[None, :, :], axis=-1)
    cbr = cb.astype(_BF).astype(_F32)
    xr = x.astype(_BF).astype(_F32)
    cross = jnp.sum(xr[:, None, :] * cbr[None, :, :], axis=-1)
    d2 = norm - 2.0 * cross
    cand = jnp.broadcast_to(cb[None, :, :], (bsz, kb, cb.shape[1]))
    _argmin_update(d2, cand, blk, kb, codes_ref, xhat_ref, best_ref)


def _steps_body(x_ref, xhat0_ref, cb_ref, w_in_ref, b_in_ref, wct_ref,
                wcb_ref, b_cat_ref, w1_ref, b1_ref, w2_ref, b2_ref,
                w_out_ref, b_out_ref, codes_ref, xh_out_ref,
                xhat_s_ref, best_ref, *, n_res):
    step = pl.program_id(0)
    blk = pl.program_id(1)
    kb = cb_ref.shape[0]
    x = x_ref[...]
    cb = cb_ref[...]
    bsz = x.shape[0]

    @pl.when(blk == 0)
    def _latch():
        xhat_s_ref[...] = jnp.where(step == 0, xhat0_ref[...], xh_out_ref[...])

    xhat = xhat_s_ref[...]

    a = _bdot(_bdot(cb, w_in_ref[0]) + b_in_ref[0], wct_ref[0]) \
        + b_cat_ref[0]
    c = _bdot(xhat, wcb_ref[0])

    h = (a[None, :, :] + c[:, None, :]).reshape(bsz * kb, -1)
    for l in range(n_res):
        t = jnp.maximum(_bdot(h, w1_ref[0, l]) + b1_ref[0, l], 0.0)
        h = h + _bdot(t, w2_ref[0, l]) + b2_ref[0, l]
    out = _bdot(h, w_out_ref[0]) + b_out_ref[0]
    out3 = out.reshape(bsz, kb, -1) + cb[None, :, :] + xhat[:, None, :]

    norm = jnp.sum(out3 * out3, axis=-1)
    xr = x.astype(_BF).astype(_F32)
    our = out3.astype(_BF).astype(_F32)
    cross = jnp.sum(our * xr[:, None, :], axis=-1)
    d2 = norm - 2.0 * cross
    _argmin_update(d2, out3, blk, kb, codes_ref, xh_out_ref, best_ref)


def kernel(x, codebook, W_in, b_in, W_cat, b_cat, W1, b1, W2, b2, W_out, b_out):
    nL = W_in.shape[0]
    Mm = nL + 1
    Kk = codebook.shape[0] // Mm
    Bb, Dd = x.shape
    DH = W_in.shape[2]
    Ll = W1.shape[1]
    KB = 64
    NKB = Kk // KB
    f32 = jnp.float32

    rep2 = lambda k: (0, 0)

    codes0, xhat0 = pl.pallas_call(
        _step0_body,
        grid=(NKB,),
        in_specs=[pl.BlockSpec((Bb, Dd), rep2),
                  pl.BlockSpec((KB, Dd), lambda k: (k, 0))],
        out_specs=(pl.BlockSpec((Bb, 1), rep2), pl.BlockSpec((Bb, Dd), rep2)),
        out_shape=(jax.ShapeDtypeStruct((Bb, 1), jnp.int32),
                   jax.ShapeDtypeStruct((Bb, Dd), f32)),
        scratch_shapes=[pltpu.VMEM((Bb, 1), f32)],
        compiler_params=pltpu.CompilerParams(
            dimension_semantics=("arbitrary",)),
    )(x, codebook[:Kk])

    c2 = lambda i, k: (0, 0)
    w3 = lambda i, k: (i, 0, 0)
    w4 = lambda i, k: (i, 0, 0, 0)

    codes_steps, xhat = pl.pallas_call(
        functools.partial(_steps_body, n_res=Ll),
        grid=(nL, NKB),
        in_specs=[
            pl.BlockSpec((Bb, Dd), c2),
            pl.BlockSpec((Bb, Dd), c2),
            pl.BlockSpec((KB, Dd),
                         lambda i, k: (NKB + i * NKB + k, 0)),
            pl.BlockSpec((1, Dd, DH), w3),
            pl.BlockSpec((1, 1, DH), w3),
            pl.BlockSpec((1, DH, DH), w3),
            pl.BlockSpec((1, Dd, DH), w3),
            pl.BlockSpec((1, 1, DH), w3),
            pl.BlockSpec((1, Ll, DH, DH), w4),
            pl.BlockSpec((1, Ll, 1, DH), w4),
            pl.BlockSpec((1, Ll, DH, DH), w4),
            pl.BlockSpec((1, Ll, 1, DH), w4),
            pl.BlockSpec((1, DH, Dd), w3),
            pl.BlockSpec((1, 1, Dd), w3),
        ],
        out_specs=(pl.BlockSpec((1, Bb, 1), lambda i, k: (i, 0, 0)),
                   pl.BlockSpec((Bb, Dd), c2)),
        out_shape=(jax.ShapeDtypeStruct((nL, Bb, 1), jnp.int32),
                   jax.ShapeDtypeStruct((Bb, Dd), f32)),
        scratch_shapes=[pltpu.VMEM((Bb, Dd), f32), pltpu.VMEM((Bb, 1), f32)],
        compiler_params=pltpu.CompilerParams(
            dimension_semantics=("arbitrary", "arbitrary")),
    )(x, xhat0, codebook,
      W_in, b_in[:, None, :], W_cat[:, :DH], W_cat[:, DH:], b_cat[:, None, :],
      W1, b1[:, :, None, :], W2, b2[:, :, None, :], W_out, b_out[:, None, :])

    codes_MB = jnp.concatenate([codes0.T, codes_steps[:, :, 0]], axis=0)
    return codes_MB, xhat


# speedup vs baseline: 3.4301x; 1.1184x over previous
import functools

import jax
import jax.numpy as jnp
from jax import lax
from jax.experimental import pallas as pl
from jax.experimental.pallas import tpu as pltpu

_BF = jnp.bfloat16
_F32 = jnp.float32
_DN = (((1,), (0,)), ((), ()))


def _bdot(a, b):
    return lax.dot_general(a.astype(_BF), b.astype(_BF), _DN,
                           preferred_element_type=_F32)


def _pick_best(d2, blk, kb, best_ref, idx_ref):
    loc_min = jnp.min(d2, axis=1, keepdims=True)
    iota = lax.broadcasted_iota(jnp.int32, d2.shape, 1)
    loc_idx = jnp.min(jnp.where(d2 == loc_min, iota, kb),
                      axis=1, keepdims=True)
    gidx = blk * kb + loc_idx

    @pl.when(blk == 0)
    def _init():
        best_ref[...] = loc_min
        idx_ref[...] = gidx

    @pl.when(blk > 0)
    def _update():
        upd = loc_min < best_ref[...]
        best_ref[...] = jnp.where(upd, loc_min, best_ref[...])
        idx_ref[...] = jnp.where(upd, gidx, idx_ref[...])


def _exact_rows(idx, cbf):
    bsz = idx.shape[0]
    iota = lax.broadcasted_iota(jnp.int32, (bsz, cbf.shape[0]), 1)
    oh = (iota == idx).astype(_BF)
    hi = cbf.astype(_BF)
    r1 = cbf - hi.astype(_F32)
    mid = r1.astype(_BF)
    lo = (r1 - mid.astype(_F32)).astype(_BF)
    g = lambda m: lax.dot_general(oh, m, _DN, preferred_element_type=_F32)
    return (g(hi) + g(mid)) + g(lo)


def _step0_body(x_ref, cb_ref, cbf_ref, codes_ref, xhat_ref,
                best_ref, idx_ref):
    blk = pl.program_id(0)
    nblk = pl.num_programs(0)
    x = x_ref[...]
    cb = cb_ref[...]
    kb = cb.shape[0]
    norm = jnp.sum(cb[None, :, :] * cb[None, :, :], axis=-1)
    cbr = cb.astype(_BF).astype(_F32)
    xr = x.astype(_BF).astype(_F32)
    cross = jnp.sum(xr[:, None, :] * cbr[None, :, :], axis=-1)
    d2 = norm - 2.0 * cross
    _pick_best(d2, blk, kb, best_ref, idx_ref)

    @pl.when(blk == nblk - 1)
    def _finish():
        idx = idx_ref[...]
        xhat_ref[...] = _exact_rows(idx, cbf_ref[...])
        codes_ref[...] = idx


def _steps_body(x_ref, xhat0_ref, cb_ref, cbf_ref, w_in_ref, b_in_ref,
                wct_ref, wcb_ref, b_cat_ref, w1_ref, b1_ref, w2_ref, b2_ref,
                w_out_ref, b_out_ref, codes_ref, xh_out_ref,
                xhat_s_ref, best_ref, idx_ref, *, n_res):
    step = pl.program_id(0)
    blk = pl.program_id(1)
    nblk = pl.num_programs(1)
    kb = cb_ref.shape[0]
    x = x_ref[...]
    cb = cb_ref[...]
    bsz = x.shape[0]

    @pl.when(blk == 0)
    def _latch():
        xhat_s_ref[...] = jnp.where(step == 0, xhat0_ref[...], xh_out_ref[...])

    xhat = xhat_s_ref[...]

    a = _bdot(_bdot(cb, w_in_ref[0]) + b_in_ref[0], wct_ref[0]) \
        + b_cat_ref[0]
    c = _bdot(xhat, wcb_ref[0])

    h = (a[None, :, :] + c[:, None, :]).reshape(bsz * kb, -1)
    for l in range(n_res):
        t = jnp.maximum(_bdot(h, w1_ref[0, l]) + b1_ref[0, l], 0.0)
        h = h + _bdot(t, w2_ref[0, l]) + b2_ref[0, l]
    out = _bdot(h, w_out_ref[0]) + b_out_ref[0]
    out3 = out.reshape(bsz, kb, -1) + cb[None, :, :] + xhat[:, None, :]

    norm = jnp.sum(out3 * out3, axis=-1)
    xr = x.astype(_BF).astype(_F32)
    our = out3.astype(_BF).astype(_F32)
    cross = jnp.sum(our * xr[:, None, :], axis=-1)
    d2 = norm - 2.0 * cross
    _pick_best(d2, blk, kb, best_ref, idx_ref)

    @pl.when(blk == nblk - 1)
    def _finish():
        idx = idx_ref[...]
        cbg = _exact_rows(idx, cbf_ref[...])
        a_r = _bdot(_bdot(cbg, w_in_ref[0]) + b_in_ref[0], wct_ref[0]) \
            + b_cat_ref[0]
        h_r = a_r + c
        for l in range(n_res):
            t_r = jnp.maximum(_bdot(h_r, w1_ref[0, l]) + b1_ref[0, l], 0.0)
            h_r = h_r + _bdot(t_r, w2_ref[0, l]) + b2_ref[0, l]
        out_r = _bdot(h_r, w_out_ref[0]) + b_out_ref[0]
        xh_out_ref[...] = (out_r + cbg) + xhat
        codes_ref[0] = idx


def kernel(x, codebook, W_in, b_in, W_cat, b_cat, W1, b1, W2, b2, W_out, b_out):
    nL = W_in.shape[0]
    Mm = nL + 1
    Kk = codebook.shape[0] // Mm
    Bb, Dd = x.shape
    DH = W_in.shape[2]
    Ll = W1.shape[1]
    KB = 64
    NKB = Kk // KB
    f32 = jnp.float32

    rep2 = lambda k: (0, 0)

    codes0, xhat0 = pl.pallas_call(
        _step0_body,
        grid=(NKB,),
        in_specs=[pl.BlockSpec((Bb, Dd), rep2),
                  pl.BlockSpec((KB, Dd), lambda k: (k, 0)),
                  pl.BlockSpec((Kk, Dd), lambda k: (0, 0))],
        out_specs=(pl.BlockSpec((Bb, 1), rep2), pl.BlockSpec((Bb, Dd), rep2)),
        out_shape=(jax.ShapeDtypeStruct((Bb, 1), jnp.int32),
                   jax.ShapeDtypeStruct((Bb, Dd), f32)),
        scratch_shapes=[pltpu.VMEM((Bb, 1), f32),
                        pltpu.VMEM((Bb, 1), jnp.int32)],
        compiler_params=pltpu.CompilerParams(
            dimension_semantics=("arbitrary",)),
    )(x, codebook[:Kk], codebook[:Kk])

    c2 = lambda i, k: (0, 0)
    w3 = lambda i, k: (i, 0, 0)
    w4 = lambda i, k: (i, 0, 0, 0)

    codes_steps, xhat = pl.pallas_call(
        functools.partial(_steps_body, n_res=Ll),
        grid=(nL, NKB),
        in_specs=[
            pl.BlockSpec((Bb, Dd), c2),
            pl.BlockSpec((Bb, Dd), c2),
            pl.BlockSpec((KB, Dd),
                         lambda i, k: (NKB + i * NKB + k, 0)),
            pl.BlockSpec((Kk, Dd), lambda i, k: (i + 1, 0)),
            pl.BlockSpec((1, Dd, DH), w3),
            pl.BlockSpec((1, 1, DH), w3),
            pl.BlockSpec((1, DH, DH), w3),
            pl.BlockSpec((1, Dd, DH), w3),
            pl.BlockSpec((1, 1, DH), w3),
            pl.BlockSpec((1, Ll, DH, DH), w4),
            pl.BlockSpec((1, Ll, 1, DH), w4),
            pl.BlockSpec((1, Ll, DH, DH), w4),
            pl.BlockSpec((1, Ll, 1, DH), w4),
            pl.BlockSpec((1, DH, Dd), w3),
            pl.BlockSpec((1, 1, Dd), w3),
        ],
        out_specs=(pl.BlockSpec((1, Bb, 1), lambda i, k: (i, 0, 0)),
                   pl.BlockSpec((Bb, Dd), c2)),
        out_shape=(jax.ShapeDtypeStruct((nL, Bb, 1), jnp.int32),
                   jax.ShapeDtypeStruct((Bb, Dd), f32)),
        scratch_shapes=[pltpu.VMEM((Bb, Dd), f32), pltpu.VMEM((Bb, 1), f32),
                        pltpu.VMEM((Bb, 1), jnp.int32)],
        compiler_params=pltpu.CompilerParams(
            dimension_semantics=("arbitrary", "arbitrary")),
    )(x, xhat0, codebook, codebook,
      W_in, b_in[:, None, :], W_cat[:, :DH], W_cat[:, DH:], b_cat[:, None, :],
      W1, b1[:, :, None, :], W2, b2[:, :, None, :], W_out, b_out[:, None, :])

    codes_MB = jnp.concatenate([codes0.T, codes_steps[:, :, 0]], axis=0)
    return codes_MB, xhat


# speedup vs baseline: 3.6655x; 1.0686x over previous
import functools

import jax
import jax.numpy as jnp
from jax import lax
from jax.experimental import pallas as pl
from jax.experimental.pallas import tpu as pltpu

_BF = jnp.bfloat16
_F32 = jnp.float32
_DN = (((1,), (0,)), ((), ()))


def _bdot(a, b):
    return lax.dot_general(a.astype(_BF), b.astype(_BF), _DN,
                           preferred_element_type=_F32)


def _pick_best(d2, blk, kb, best_ref, idx_ref):
    loc_min = jnp.min(d2, axis=1, keepdims=True)
    iota = lax.broadcasted_iota(jnp.int32, d2.shape, 1)
    loc_idx = jnp.min(jnp.where(d2 == loc_min, iota, kb),
                      axis=1, keepdims=True)
    gidx = blk * kb + loc_idx

    @pl.when(blk == 0)
    def _init():
        best_ref[...] = loc_min
        idx_ref[...] = gidx

    @pl.when(blk > 0)
    def _update():
        upd = loc_min < best_ref[...]
        best_ref[...] = jnp.where(upd, loc_min, best_ref[...])
        idx_ref[...] = jnp.where(upd, gidx, idx_ref[...])


def _exact_rows(idx, cbf):
    bsz = idx.shape[0]
    iota = lax.broadcasted_iota(jnp.int32, (bsz, cbf.shape[0]), 1)
    oh = (iota == idx).astype(_BF)
    hi = cbf.astype(_BF)
    r1 = cbf - hi.astype(_F32)
    mid = r1.astype(_BF)
    lo = (r1 - mid.astype(_F32)).astype(_BF)
    g = lambda m: lax.dot_general(oh, m, _DN, preferred_element_type=_F32)
    return (g(hi) + g(mid)) + g(lo)


def _step0_body(x_ref, cb_ref, cbf_ref, codes_ref, xhat_ref,
                best_ref, idx_ref):
    blk = pl.program_id(0)
    nblk = pl.num_programs(0)
    x = x_ref[...]
    cb = cb_ref[...]
    kb = cb.shape[0]
    norm = jnp.sum(cb[None, :, :] * cb[None, :, :], axis=-1)
    cbr = cb.astype(_BF).astype(_F32)
    xr = x.astype(_BF).astype(_F32)
    cross = jnp.sum(xr[:, None, :] * cbr[None, :, :], axis=-1)
    d2 = norm - 2.0 * cross
    _pick_best(d2, blk, kb, best_ref, idx_ref)

    @pl.when(blk == nblk - 1)
    def _finish():
        idx = idx_ref[...]
        xhat_ref[...] = _exact_rows(idx, cbf_ref[...])
        codes_ref[...] = idx


def _steps_body(x_ref, xhat0_ref, cb_ref, cbf_ref, w_in_ref, b_in_ref,
                wct_ref, wcb_ref, b_cat_ref, w1_ref, b1_ref, w2_ref, b2_ref,
                w_out_ref, b_out_ref, codes_ref, xh_out_ref,
                xhat_s_ref, best_ref, idx_ref, *, n_res):
    step = pl.program_id(0)
    blk = pl.program_id(1)
    nblk = pl.num_programs(1)
    kb = cb_ref.shape[0]
    x = x_ref[...]
    cb = cb_ref[...]
    bsz = x.shape[0]

    @pl.when(blk == 0)
    def _latch():
        xhat_s_ref[...] = jnp.where(step == 0, xhat0_ref[...], xh_out_ref[...])

    xhat = xhat_s_ref[...]

    a = _bdot(_bdot(cb, w_in_ref[0]) + b_in_ref[0], wct_ref[0]) \
        + b_cat_ref[0]
    c = _bdot(xhat, wcb_ref[0])

    h = (a[None, :, :] + c[:, None, :]).reshape(bsz * kb, -1)
    for l in range(n_res):
        t = jnp.maximum(_bdot(h, w1_ref[0, l]) + b1_ref[0, l], 0.0)
        h = h + _bdot(t, w2_ref[0, l]) + b2_ref[0, l]
    out = _bdot(h, w_out_ref[0]) + b_out_ref[0]
    out3 = out.reshape(bsz, kb, -1) + cb[None, :, :] + xhat[:, None, :]

    norm = jnp.sum(out3 * out3, axis=-1)
    xr = x.astype(_BF).astype(_F32)
    our = out3.astype(_BF).astype(_F32)
    cross = jnp.sum(our * xr[:, None, :], axis=-1)
    d2 = norm - 2.0 * cross
    _pick_best(d2, blk, kb, best_ref, idx_ref)

    @pl.when(blk == nblk - 1)
    def _finish():
        idx = idx_ref[...]
        cbg = _exact_rows(idx, cbf_ref[...])
        a_r = _bdot(_bdot(cbg, w_in_ref[0]) + b_in_ref[0], wct_ref[0]) \
            + b_cat_ref[0]
        h_r = a_r + c
        for l in range(n_res):
            t_r = jnp.maximum(_bdot(h_r, w1_ref[0, l]) + b1_ref[0, l], 0.0)
            h_r = h_r + _bdot(t_r, w2_ref[0, l]) + b2_ref[0, l]
        out_r = _bdot(h_r, w_out_ref[0]) + b_out_ref[0]
        xh_out_ref[...] = (out_r + cbg) + xhat
        codes_ref[0] = idx


def kernel(x, codebook, W_in, b_in, W_cat, b_cat, W1, b1, W2, b2, W_out, b_out):
    nL = W_in.shape[0]
    Mm = nL + 1
    Kk = codebook.shape[0] // Mm
    Bb, Dd = x.shape
    DH = W_in.shape[2]
    Ll = W1.shape[1]
    KB = 128
    NKB = Kk // KB
    f32 = jnp.float32

    rep2 = lambda k: (0, 0)

    codes0, xhat0 = pl.pallas_call(
        _step0_body,
        grid=(NKB,),
        in_specs=[pl.BlockSpec((Bb, Dd), rep2),
                  pl.BlockSpec((KB, Dd), lambda k: (k, 0)),
                  pl.BlockSpec((Kk, Dd), lambda k: (0, 0))],
        out_specs=(pl.BlockSpec((Bb, 1), rep2), pl.BlockSpec((Bb, Dd), rep2)),
        out_shape=(jax.ShapeDtypeStruct((Bb, 1), jnp.int32),
                   jax.ShapeDtypeStruct((Bb, Dd), f32)),
        scratch_shapes=[pltpu.VMEM((Bb, 1), f32),
                        pltpu.VMEM((Bb, 1), jnp.int32)],
        compiler_params=pltpu.CompilerParams(
            dimension_semantics=("arbitrary",)),
    )(x, codebook[:Kk], codebook[:Kk])

    c2 = lambda i, k: (0, 0)
    w3 = lambda i, k: (i, 0, 0)
    w4 = lambda i, k: (i, 0, 0, 0)

    codes_steps, xhat = pl.pallas_call(
        functools.partial(_steps_body, n_res=Ll),
        grid=(nL, NKB),
        in_specs=[
            pl.BlockSpec((Bb, Dd), c2),
            pl.BlockSpec((Bb, Dd), c2),
            pl.BlockSpec((KB, Dd),
                         lambda i, k: (NKB + i * NKB + k, 0)),
            pl.BlockSpec((Kk, Dd), lambda i, k: (i + 1, 0)),
            pl.BlockSpec((1, Dd, DH), w3),
            pl.BlockSpec((1, 1, DH), w3),
            pl.BlockSpec((1, DH, DH), w3),
            pl.BlockSpec((1, Dd, DH), w3),
            pl.BlockSpec((1, 1, DH), w3),
            pl.BlockSpec((1, Ll, DH, DH), w4),
            pl.BlockSpec((1, Ll, 1, DH), w4),
            pl.BlockSpec((1, Ll, DH, DH), w4),
            pl.BlockSpec((1, Ll, 1, DH), w4),
            pl.BlockSpec((1, DH, Dd), w3),
            pl.BlockSpec((1, 1, Dd), w3),
        ],
        out_specs=(pl.BlockSpec((1, Bb, 1), lambda i, k: (i, 0, 0)),
                   pl.BlockSpec((Bb, Dd), c2)),
        out_shape=(jax.ShapeDtypeStruct((nL, Bb, 1), jnp.int32),
                   jax.ShapeDtypeStruct((Bb, Dd), f32)),
        scratch_shapes=[pltpu.VMEM((Bb, Dd), f32), pltpu.VMEM((Bb, 1), f32),
                        pltpu.VMEM((Bb, 1), jnp.int32)],
        compiler_params=pltpu.CompilerParams(
            dimension_semantics=("arbitrary", "arbitrary")),
    )(x, xhat0, codebook, codebook,
      W_in, b_in[:, None, :], W_cat[:, :DH], W_cat[:, DH:], b_cat[:, None, :],
      W1, b1[:, :, None, :], W2, b2[:, :, None, :], W_out, b_out[:, None, :])

    codes_MB = jnp.concatenate([codes0.T, codes_steps[:, :, 0]], axis=0)
    return codes_MB, xhat
